# Initial kernel scaffold; baseline (speedup 1.0000x reference)
import jax
import jax.numpy as jnp
from jax.experimental import pallas as pl


def kernel(x, pos, Wh1, bh1, Wh2, bh2, Wf, bf, Wg1, bg1, Wg2, bg2, edge_index):
    raise NotImplementedError("write your pallas kernel here")


# speedup vs baseline: 7.1792x; 7.1792x over previous
import functools

import jax
import jax.numpy as jnp
from jax import lax
from jax.experimental import pallas as pl
from jax.experimental.pallas import tpu as pltpu
from jax.experimental.pallas import tpu_sc as plsc

NC = 2
NS = 16
NW = NC * NS
LANES = 16
CH = 128
RB = 512
ZR = 64


def _prologue_body(x_ref, pos_ref, wh1_ref, bh1_ref, wh2_ref, bh2_ref,
                   wfx_ref, wfp_ref, bf_ref, a_ref, b_ref):
    xb = x_ref[...]
    pb = pos_ref[...]
    h = jnp.maximum(
        jnp.dot(xb, wh1_ref[...], preferred_element_type=jnp.float32)
        + bh1_ref[...], 0.0)
    dl = jnp.tanh(
        jnp.dot(h, wh2_ref[...], preferred_element_type=jnp.float32)
        + bh2_ref[...])
    a_ref[...] = (
        jnp.dot(xb, wfx_ref[...], preferred_element_type=jnp.float32)
        + jnp.dot(pb, wfp_ref[...], preferred_element_type=jnp.float32))
    b_ref[...] = (
        jnp.dot(dl - pb, wfp_ref[...], preferred_element_type=jnp.float32)
        + bf_ref[...])


def _epilogue_body(parts_ref, x_ref, wg1_ref, bg1_ref, wg2_ref, bg2_ref,
                   out_ref):
    agg = parts_ref[0] + parts_ref[1]
    g = jnp.maximum(
        jnp.dot(agg, wg1_ref[...], preferred_element_type=jnp.float32)
        + bg1_ref[...], 0.0)
    o = jnp.maximum(
        jnp.dot(g, wg2_ref[...], preferred_element_type=jnp.float32)
        + bg2_ref[...], 0.0)
    out_ref[...] = x_ref[...] + o


def _sc_edge_body(npad, tpw, a_hbm, b_hbm, src_hbm, dst_hbm, out_hbm,
                  sidx, didx, arows, brows, zrows, acc, sga, sgb):
    c = lax.axis_index("c")
    s = lax.axis_index("s")
    w = s * NC + c

    zero16 = jnp.zeros((LANES,), jnp.float32)

    def zfill(i, carry):
        r = i // 8
        cc = i % 8
        zrows[r, pl.ds(cc * LANES, LANES)] = zero16
        return carry

    lax.fori_loop(0, ZR * 8, zfill, 0)

    rpt = npad // NS
    rbase = s * rpt

    def zinit(k, carry):
        pltpu.sync_copy(zrows, acc.at[pl.ds(rbase + k * ZR, ZR)])
        return carry

    lax.fori_loop(0, rpt // ZR, zinit, 0)
    plsc.subcore_barrier()

    def chunk(t, carry):
        base = (w * tpw + t) * CH
        pltpu.sync_copy(src_hbm.at[pl.ds(base, CH)], sidx)
        pltpu.sync_copy(dst_hbm.at[pl.ds(base, CH)], didx)
        da = pltpu.async_copy(a_hbm.at[sidx], arows, sga)
        db = pltpu.async_copy(b_hbm.at[didx], brows, sgb)
        da.wait()
        db.wait()

        def row(r, carry2):
            for cc in range(8):
                sl = pl.ds(cc * LANES, LANES)
                brows[r, sl] = jnp.maximum(arows[r, sl] + brows[r, sl], 0.0)
            return carry2

        lax.fori_loop(0, CH, row, 0)
        pltpu.sync_copy(brows, acc.at[didx], add=True)
        return carry

    lax.fori_loop(0, tpw, chunk, 0)
    plsc.subcore_barrier()
    pltpu.sync_copy(acc.at[pl.ds(rbase, rpt)],
                    out_hbm.at[c].at[pl.ds(rbase, rpt)])


def kernel(x, pos, Wh1, bh1, Wh2, bh2, Wf, bf, Wg1, bg1, Wg2, bg2, edge_index):
    n, d = x.shape
    e = edge_index.shape[1]
    npad = ((n + NS * ZR - 1) // (NS * ZR)) * (NS * ZR)
    tpw = (e + NW * CH - 1) // (NW * CH)
    epad = NW * CH * tpw

    xp = jnp.pad(x, ((0, npad - n), (0, 0)))
    posp = jnp.pad(pos, ((0, npad - n), (0, d - pos.shape[1])))
    wfp = jnp.pad(Wf[:3], ((0, d - 3), (0, 0)))
    wfx = Wf[3:]
    wh2p = jnp.pad(Wh2, ((0, 0), (0, d - Wh2.shape[1])))
    bh2p = jnp.pad(bh2, (0, d - bh2.shape[0])).reshape(1, d)
    bh1r = bh1.reshape(1, d)
    bfr = bf.reshape(1, d)
    bg1r = bg1.reshape(1, -1)
    bg2r = bg2.reshape(1, -1)
    src = jnp.pad(edge_index[0], (0, epad - e))
    dst = jnp.pad(edge_index[1], (0, epad - e),
                  constant_values=n)

    grid1 = (npad // RB,)
    row_spec = pl.BlockSpec((RB, d), lambda i: (i, 0))
    w_spec = pl.BlockSpec((d, d), lambda i: (0, 0))
    bias_spec = pl.BlockSpec((1, d), lambda i: (0, 0))
    a_tab, b_tab = pl.pallas_call(
        _prologue_body,
        grid=grid1,
        in_specs=[row_spec, row_spec, w_spec, bias_spec, w_spec, bias_spec,
                  w_spec, w_spec, bias_spec],
        out_specs=[row_spec, row_spec],
        out_shape=[jax.ShapeDtypeStruct((npad, d), jnp.float32),
                   jax.ShapeDtypeStruct((npad, d), jnp.float32)],
    )(xp, posp, Wh1, bh1r, wh2p, bh2p, wfx, wfp, bfr)

    mesh = plsc.VectorSubcoreMesh(core_axis_name="c", subcore_axis_name="s",
                                  num_cores=NC, num_subcores=NS)
    sc_edge = functools.partial(
        pl.kernel,
        out_type=jax.ShapeDtypeStruct((NC, npad, d), jnp.float32),
        mesh=mesh,
        scratch_types=[
            pltpu.VMEM((CH,), jnp.int32),
            pltpu.VMEM((CH,), jnp.int32),
            pltpu.VMEM((CH, d), jnp.float32),
            pltpu.VMEM((CH, d), jnp.float32),
            pltpu.VMEM((ZR, d), jnp.float32),
            pltpu.VMEM_SHARED((npad, d), jnp.float32),
            pltpu.SemaphoreType.DMA,
            pltpu.SemaphoreType.DMA,
        ],
    )(functools.partial(_sc_edge_body, npad, tpw))
    parts = sc_edge(a_tab, b_tab, src, dst)

    rbe = 1000 if n % 1000 == 0 else RB
    grid3 = ((n + rbe - 1) // rbe,)
    parts_spec = pl.BlockSpec((NC, rbe, d), lambda i: (0, i, 0))
    row_spec_e = pl.BlockSpec((rbe, d), lambda i: (i, 0))
    out = pl.pallas_call(
        _epilogue_body,
        grid=grid3,
        in_specs=[parts_spec, row_spec_e, w_spec, bias_spec, w_spec, bias_spec],
        out_specs=row_spec_e,
        out_shape=jax.ShapeDtypeStruct((n, d), jnp.float32),
    )(parts, x, Wg1, bg1r, Wg2, bg2r)
    return out


# speedup vs baseline: 10.4578x; 1.4567x over previous
import functools

import jax
import jax.numpy as jnp
from jax import lax
from jax.experimental import pallas as pl
from jax.experimental.pallas import tpu as pltpu
from jax.experimental.pallas import tpu_sc as plsc

NC = 2
NS = 16
NW = NC * NS
LANES = 16
CH = 48
NPH = 2
RB = 512
ZR = 40


def _prologue_body(x_ref, pos_ref, wh1_ref, bh1_ref, wh2_ref, bh2_ref,
                   wfx_ref, wfp_ref, bf_ref, a_ref, b_ref):
    xb = x_ref[...]
    pb = pos_ref[...]
    h = jnp.maximum(
        jnp.dot(xb, wh1_ref[...], preferred_element_type=jnp.float32)
        + bh1_ref[...], 0.0)
    dl = jnp.tanh(
        jnp.dot(h, wh2_ref[...], preferred_element_type=jnp.float32)
        + bh2_ref[...])
    a_ref[...] = (
        jnp.dot(xb, wfx_ref[...], preferred_element_type=jnp.float32)
        + jnp.dot(pb, wfp_ref[...], preferred_element_type=jnp.float32))
    b_ref[...] = (
        jnp.dot(dl - pb, wfp_ref[...], preferred_element_type=jnp.float32)
        + bf_ref[...])


def _epilogue_body(parts_ref, x_ref, wg1_ref, bg1_ref, wg2_ref, bg2_ref,
                   out_ref):
    agg = parts_ref[0] + parts_ref[1]
    g = jnp.maximum(
        jnp.dot(agg, wg1_ref[...], preferred_element_type=jnp.float32)
        + bg1_ref[...], 0.0)
    o = jnp.maximum(
        jnp.dot(g, wg2_ref[...], preferred_element_type=jnp.float32)
        + bg2_ref[...], 0.0)
    out_ref[...] = x_ref[...] + o


def _sc_edge_body(npad, tpw2, a_hbm, b_hbm, src_hbm, dst_hbm, out_hbm,
                  sidx_all, didx_all, ar0, br0, ar1, br1, ar2, br2,
                  di0, di1, di2, acc,
                  ga0, gb0, ga1, gb1, ga2, gb2, sc0, sc1, sc2):
    c = lax.axis_index("c")
    s = lax.axis_index("s")
    w = s * NC + c
    sets = ((ar0, br0, di0, ga0, gb0, sc0),
            (ar1, br1, di1, ga1, gb1, sc1),
            (ar2, br2, di2, ga2, gb2, sc2))
    nsl = ar0.shape[1] // LANES
    isl = CH // LANES

    zero16 = jnp.zeros((LANES,), jnp.float32)

    def zfill(i, carry):
        r = i // nsl
        cc = i % nsl
        ar0[r, pl.ds(cc * LANES, LANES)] = zero16
        return carry

    lax.fori_loop(0, ZR * nsl, zfill, 0)

    rpt = npad // NS
    rbase = s * rpt

    def zinit(k, carry):
        pltpu.sync_copy(ar0.at[pl.ds(0, ZR)],
                        acc.at[pl.ds(rbase + k * ZR, ZR)])
        return carry

    lax.fori_loop(0, rpt // ZR, zinit, 0)
    plsc.subcore_barrier()

    def issue(ch, st):
        ar, br, _, ga, gb, _sc = st
        pltpu.async_copy(a_hbm.at[sidx_all.at[pl.ds(ch * CH, CH)]], ar, ga)
        pltpu.async_copy(b_hbm.at[didx_all.at[pl.ds(ch * CH, CH)]], br, gb)

    def gwait(ch, st):
        ar, br, _, ga, gb, _sc = st
        pltpu.make_async_copy(
            a_hbm.at[sidx_all.at[pl.ds(ch * CH, CH)]], ar, ga).wait()
        pltpu.make_async_copy(
            b_hbm.at[didx_all.at[pl.ds(ch * CH, CH)]], br, gb).wait()

    for ph in range(NPH):
        pltpu.sync_copy(src_hbm.at[w, ph], sidx_all)
        pltpu.sync_copy(dst_hbm.at[w, ph], didx_all)

        issue(0, sets[0])
        issue(1, sets[1])

        def iter3(i, carry):
            for k in range(3):
                ar, br, di, ga, gb, sc = sets[k]
                ch = 3 * i + k
                gwait(ch, sets[k])

                for cc in range(isl):
                    di[pl.ds(cc * LANES, LANES)] = (
                        didx_all[pl.ds(ch * CH + cc * LANES, LANES)])

                def row(r, carry2):
                    for cc in range(nsl):
                        sl = pl.ds(cc * LANES, LANES)
                        br[r, sl] = jnp.maximum(ar[r, sl] + br[r, sl], 0.0)
                    return carry2

                lax.fori_loop(0, CH, row, 0)
                pltpu.async_copy(br, acc.at[di], sc, add=True)

                j = (k + 2) % 3
                brj, dij, scj = sets[j][1], sets[j][2], sets[j][5]

                @pl.when(ch > 0)
                def _():
                    pltpu.make_async_copy(brj, acc.at[dij], scj).wait()

                @pl.when(ch + 2 < tpw2)
                def _():
                    issue(ch + 2, sets[j])
            return carry

        lax.fori_loop(0, tpw2 // 3, iter3, 0)
        pltpu.make_async_copy(
            sets[2][1], acc.at[sets[2][2]], sets[2][5]).wait()

    plsc.subcore_barrier()
    pltpu.sync_copy(acc.at[pl.ds(rbase, rpt)],
                    out_hbm.at[c].at[pl.ds(rbase, rpt)])


def kernel(x, pos, Wh1, bh1, Wh2, bh2, Wf, bf, Wg1, bg1, Wg2, bg2, edge_index):
    n, d = x.shape
    e = edge_index.shape[1]
    npad = ((n + NS * ZR - 1) // (NS * ZR)) * (NS * ZR)
    tpw = (e + NW * CH - 1) // (NW * CH)
    tpw = ((tpw + 3 * NPH - 1) // (3 * NPH)) * (3 * NPH)
    tpw2 = tpw // NPH
    epad = NW * CH * tpw

    xp = jnp.pad(x, ((0, npad - n), (0, 0)))
    posp = jnp.pad(pos, ((0, npad - n), (0, d - pos.shape[1])))
    wfp = jnp.pad(Wf[:3], ((0, d - 3), (0, 0)))
    wfx = Wf[3:]
    wh2p = jnp.pad(Wh2, ((0, 0), (0, d - Wh2.shape[1])))
    bh2p = jnp.pad(bh2, (0, d - bh2.shape[0])).reshape(1, d)
    bh1r = bh1.reshape(1, d)
    bfr = bf.reshape(1, d)
    bg1r = bg1.reshape(1, -1)
    bg2r = bg2.reshape(1, -1)
    src = jnp.pad(edge_index[0], (0, epad - e)).reshape(NW, NPH, tpw2 * CH)
    dst = jnp.pad(edge_index[1], (0, epad - e),
                  constant_values=n).reshape(NW, NPH, tpw2 * CH)

    grid1 = (npad // RB,)
    row_spec = pl.BlockSpec((RB, d), lambda i: (i, 0))
    w_spec = pl.BlockSpec((d, d), lambda i: (0, 0))
    bias_spec = pl.BlockSpec((1, d), lambda i: (0, 0))
    a_tab, b_tab = pl.pallas_call(
        _prologue_body,
        grid=grid1,
        in_specs=[row_spec, row_spec, w_spec, bias_spec, w_spec, bias_spec,
                  w_spec, w_spec, bias_spec],
        out_specs=[row_spec, row_spec],
        out_shape=[jax.ShapeDtypeStruct((npad, d), jnp.float32)] * 2,
    )(xp, posp, Wh1, bh1r, wh2p, bh2p, wfx, wfp, bfr)

    mesh = plsc.VectorSubcoreMesh(core_axis_name="c", subcore_axis_name="s",
                                  num_cores=NC, num_subcores=NS)
    sc_edge = functools.partial(
        pl.kernel,
        out_type=jax.ShapeDtypeStruct((NC, npad, d), jnp.float32),
        mesh=mesh,
        scratch_types=(
            [pltpu.VMEM((tpw2 * CH,), jnp.int32)] * 2
            + [pltpu.VMEM((CH, d), jnp.float32)] * 6
            + [pltpu.VMEM((CH,), jnp.int32)] * 3
            + [pltpu.VMEM_SHARED((npad, d), jnp.float32)]
            + [pltpu.SemaphoreType.DMA] * 9
        ),
    )(functools.partial(_sc_edge_body, npad, tpw2))
    parts = sc_edge(a_tab, b_tab, src, dst)

    rbe = 1000 if n % 1000 == 0 else RB
    grid3 = ((n + rbe - 1) // rbe,)
    parts_spec = pl.BlockSpec((NC, rbe, d), lambda i: (0, i, 0))
    row_spec_e = pl.BlockSpec((rbe, d), lambda i: (i, 0))
    out = pl.pallas_call(
        _epilogue_body,
        grid=grid3,
        in_specs=[parts_spec, row_spec_e, w_spec, bias_spec, w_spec,
                  bias_spec],
        out_specs=row_spec_e,
        out_shape=jax.ShapeDtypeStruct((n, d), jnp.float32),
    )(parts, x, Wg1, bg1r, Wg2, bg2r)
    return out
